# baseline (speedup 1.0000x reference)
import functools

import jax
import jax.numpy as jnp
from jax import lax
from jax.experimental import pallas as pl
from jax.experimental.pallas import tpu as pltpu
from jax.experimental.pallas import tpu_sc as plsc

B = 2048
D = 256
K = 1024
ND = 8
H = D // 2

BM = 256
NBLK = B // BM

NW = 32
RPW = B // NW


def _gelu_exact(x):
    return 0.5 * x * (1.0 + lax.erf(x * (2.0 ** -0.5)))


def _layernorm(x, g, b):
    mu = jnp.mean(x, axis=-1, keepdims=True)
    var = jnp.mean((x - mu) ** 2, axis=-1, keepdims=True)
    return (x - mu) / jnp.sqrt(var + 1e-5) * g + b


def _heads_body(x_ref, w1_ref, b1_ref, g1_ref, be1_ref, w2_ref, b2_ref,
                pw1_ref, pb1_ref, pg_ref, pbe_ref, pw2_ref, pb2_ref,
                feats_ref, w_ref):
    x = x_ref[...]
    dot = functools.partial(jnp.dot, preferred_element_type=jnp.float32,
                            precision=lax.Precision.HIGHEST)
    h = dot(x, w1_ref[...]) + b1_ref[...]
    h = _layernorm(h, g1_ref[...], be1_ref[...])
    h = _gelu_exact(h)
    h = dot(h, w2_ref[...]) + b2_ref[...]
    n = jnp.sqrt(jnp.sum(h * h, axis=-1, keepdims=True))
    feats_ref[...] = h / jnp.maximum(n, 1e-12)
    t = dot(x, pw1_ref[...]) + pb1_ref[...]
    t = _layernorm(t, pg_ref[...], pbe_ref[...])
    t = _gelu_exact(t)
    logits = dot(t, pw2_ref[...]) + pb2_ref[...]
    m = jnp.max(logits, axis=-1, keepdims=True)
    e = jnp.exp(logits - m)
    w_ref[...] = e / jnp.sum(e, axis=-1, keepdims=True)


def _run_heads(features, ph_W1, ph_b1, ln1_g, ln1_b, ph_W2, ph_b2,
               pw_W1, pw_b1, pw_ln_g, pw_ln_b, pw_W2, pw_b2):
    def const(shape):
        return pl.BlockSpec(shape, lambda i: (0,) * len(shape))
    return pl.pallas_call(
        _heads_body,
        grid=(NBLK,),
        in_specs=[
            pl.BlockSpec((BM, D), lambda i: (i, 0)),
            const((D, D)), const((1, D)), const((1, D)), const((1, D)),
            const((D, D)), const((1, D)),
            const((D, H)), const((1, H)), const((1, H)), const((1, H)),
            const((H, K)), const((1, K)),
        ],
        out_specs=[
            pl.BlockSpec((BM, D), lambda i: (i, 0)),
            pl.BlockSpec((BM, K), lambda i: (i, 0)),
        ],
        out_shape=[
            jax.ShapeDtypeStruct((B, D), jnp.float32),
            jax.ShapeDtypeStruct((B, K), jnp.float32),
        ],
    )(features, ph_W1, ph_b1.reshape(1, D), ln1_g.reshape(1, D),
      ln1_b.reshape(1, D), ph_W2, ph_b2.reshape(1, D),
      pw_W1, pw_b1.reshape(1, H), pw_ln_g.reshape(1, H),
      pw_ln_b.reshape(1, H), pw_W2, pw_b2.reshape(1, K))


def _mix_body(offs_ref, w_ref, f_ref, p_ref, g_ref, out_ref):
    i = pl.program_id(0)
    lo = i * BM
    W = w_ref[...]
    G = g_ref[...]
    rows = lo + lax.broadcasted_iota(jnp.int32, (BM, 1), 0)
    out_ref[...] = f_ref[...]
    for n in range(ND):
        s = offs_ref[n]
        e = offs_ref[n + 1]

        @pl.when((e > lo) & (s < lo + BM))
        def _():
            mask = ((rows >= s) & (rows < e)).astype(jnp.float32)
            Mn = 0.6 * p_ref[pl.ds(n * K, K), :] + 0.4 * G
            out_ref[...] += 0.2 * jnp.dot(
                W * mask, Mn, preferred_element_type=jnp.float32,
                precision=lax.Precision.HIGHEST)


def _run_mix(offs, w_sorted, f_sorted, p_flat, g):
    grid_spec = pltpu.PrefetchScalarGridSpec(
        num_scalar_prefetch=1,
        grid=(NBLK,),
        in_specs=[
            pl.BlockSpec((BM, K), lambda i, offs: (i, 0)),
            pl.BlockSpec((BM, D), lambda i, offs: (i, 0)),
            pl.BlockSpec((ND * K, D), lambda i, offs: (0, 0)),
            pl.BlockSpec((K, D), lambda i, offs: (0, 0)),
        ],
        out_specs=pl.BlockSpec((BM, D), lambda i, offs: (i, 0)),
    )
    return pl.pallas_call(
        _mix_body,
        grid_spec=grid_spec,
        out_shape=jax.ShapeDtypeStruct((B, D), jnp.float32),
    )(offs, w_sorted, f_sorted, p_flat, g)


@functools.cache
def _sc_gather():
    mesh = plsc.VectorSubcoreMesh(core_axis_name="c", subcore_axis_name="s")

    @functools.partial(
        pl.kernel, mesh=mesh,
        out_type=(jax.ShapeDtypeStruct((B, K), jnp.float32),
                  jax.ShapeDtypeStruct((B, D), jnp.float32)),
        scratch_types=[pltpu.VMEM((RPW,), jnp.int32),
                       pltpu.VMEM((RPW, K), jnp.float32),
                       pltpu.VMEM((RPW, D), jnp.float32),
                       pltpu.SemaphoreType.DMA],
    )
    def gather(w_hbm, f_hbm, perm_hbm, wout_hbm, fout_hbm,
               idx_v, wrows_v, frows_v, sem):
        wid = lax.axis_index("s") * 2 + lax.axis_index("c")
        base = wid * RPW
        pltpu.sync_copy(perm_hbm.at[pl.ds(base, RPW)], idx_v)
        pltpu.async_copy(w_hbm.at[idx_v], wrows_v, sem).wait()
        pltpu.sync_copy(wrows_v, wout_hbm.at[pl.ds(base, RPW)])
        pltpu.async_copy(f_hbm.at[idx_v], frows_v, sem).wait()
        pltpu.sync_copy(frows_v, fout_hbm.at[pl.ds(base, RPW)])

    return gather


@functools.cache
def _sc_scatter():
    mesh = plsc.VectorSubcoreMesh(core_axis_name="c", subcore_axis_name="s")

    @functools.partial(
        pl.kernel, mesh=mesh,
        out_type=jax.ShapeDtypeStruct((B, D), jnp.float32),
        scratch_types=[pltpu.VMEM((RPW,), jnp.int32),
                       pltpu.VMEM((RPW, D), jnp.float32),
                       pltpu.SemaphoreType.DMA],
    )
    def scatter(enh_hbm, perm_hbm, out_hbm, idx_v, rows_v, sem):
        wid = lax.axis_index("s") * 2 + lax.axis_index("c")
        base = wid * RPW
        pltpu.sync_copy(perm_hbm.at[pl.ds(base, RPW)], idx_v)
        pltpu.sync_copy(enh_hbm.at[pl.ds(base, RPW)], rows_v)
        pltpu.async_copy(rows_v, out_hbm.at[idx_v], sem).wait()

    return scatter


def kernel(features, domain_ids, ph_W1, ph_b1, ln1_g, ln1_b, ph_W2, ph_b2,
           pw_W1, pw_b1, pw_ln_g, pw_ln_b, pw_W2, pw_b2,
           domain_prototypes, global_prototypes):
    did = jnp.minimum(domain_ids, ND - 1).astype(jnp.int32)
    perm = jnp.argsort(did).astype(jnp.int32)
    counts = jnp.bincount(did, length=ND)
    offs = jnp.concatenate(
        [jnp.zeros((1,), jnp.int32), jnp.cumsum(counts).astype(jnp.int32)])

    feats, w = _run_heads(features, ph_W1, ph_b1, ln1_g, ln1_b, ph_W2, ph_b2,
                          pw_W1, pw_b1, pw_ln_g, pw_ln_b, pw_W2, pw_b2)
    w_sorted, f_sorted = _sc_gather()(w, feats, perm)
    enh_sorted = _run_mix(offs, w_sorted, f_sorted,
                          domain_prototypes.reshape(ND * K, D),
                          global_prototypes)
    enhanced = _sc_scatter()(enh_sorted, perm)
    return (enhanced, w)


# speedup vs baseline: 1.3042x; 1.3042x over previous
import functools

import jax
import jax.numpy as jnp
from jax import lax
from jax.experimental import pallas as pl
from jax.experimental.pallas import tpu as pltpu
from jax.experimental.pallas import tpu_sc as plsc

B = 2048
D = 256
K = 1024
ND = 8
H = D // 2

BM = 256
NBLK = B // BM

NW = 32
RPW = B // NW


def _gelu_exact(x):
    return 0.5 * x * (1.0 + lax.erf(x * (2.0 ** -0.5)))


def _layernorm(x, g, b):
    mu = jnp.mean(x, axis=-1, keepdims=True)
    var = jnp.mean((x - mu) ** 2, axis=-1, keepdims=True)
    return (x - mu) / jnp.sqrt(var + 1e-5) * g + b


def _heads_body(x_ref, w1_ref, b1_ref, g1_ref, be1_ref, w2_ref, b2_ref,
                pw1_ref, pb1_ref, pg_ref, pbe_ref, pw2_ref, pb2_ref,
                feats_ref, w_ref):
    x = x_ref[...]
    dot = functools.partial(jnp.dot, preferred_element_type=jnp.float32)
    h = dot(x, w1_ref[...]) + b1_ref[...]
    h = _layernorm(h, g1_ref[...], be1_ref[...])
    h = _gelu_exact(h)
    h = dot(h, w2_ref[...]) + b2_ref[...]
    n = jnp.sqrt(jnp.sum(h * h, axis=-1, keepdims=True))
    feats_ref[...] = h / jnp.maximum(n, 1e-12)
    t = dot(x, pw1_ref[...]) + pb1_ref[...]
    t = _layernorm(t, pg_ref[...], pbe_ref[...])
    t = _gelu_exact(t)
    logits = dot(t, pw2_ref[...]) + pb2_ref[...]
    m = jnp.max(logits, axis=-1, keepdims=True)
    e = jnp.exp(logits - m)
    w_ref[...] = e / jnp.sum(e, axis=-1, keepdims=True)


def _run_heads(features, ph_W1, ph_b1, ln1_g, ln1_b, ph_W2, ph_b2,
               pw_W1, pw_b1, pw_ln_g, pw_ln_b, pw_W2, pw_b2):
    def const(shape):
        return pl.BlockSpec(shape, lambda i: (0,) * len(shape))
    return pl.pallas_call(
        _heads_body,
        grid=(NBLK,),
        in_specs=[
            pl.BlockSpec((BM, D), lambda i: (i, 0)),
            const((D, D)), const((1, D)), const((1, D)), const((1, D)),
            const((D, D)), const((1, D)),
            const((D, H)), const((1, H)), const((1, H)), const((1, H)),
            const((H, K)), const((1, K)),
        ],
        out_specs=[
            pl.BlockSpec((BM, D), lambda i: (i, 0)),
            pl.BlockSpec((BM, K), lambda i: (i, 0)),
        ],
        out_shape=[
            jax.ShapeDtypeStruct((B, D), jnp.float32),
            jax.ShapeDtypeStruct((B, K), jnp.float32),
        ],
    )(features, ph_W1, ph_b1.reshape(1, D), ln1_g.reshape(1, D),
      ln1_b.reshape(1, D), ph_W2, ph_b2.reshape(1, D),
      pw_W1, pw_b1.reshape(1, H), pw_ln_g.reshape(1, H),
      pw_ln_b.reshape(1, H), pw_W2, pw_b2.reshape(1, K))


def _mix_body(offs_ref, w_ref, f_ref, p_ref, g_ref, out_ref):
    i = pl.program_id(0)
    lo = i * BM
    W = w_ref[...]
    G = g_ref[...]
    rows = lo + lax.broadcasted_iota(jnp.int32, (BM, 1), 0)
    out_ref[...] = f_ref[...]
    for n in range(ND):
        s = offs_ref[n]
        e = offs_ref[n + 1]

        @pl.when((e > lo) & (s < lo + BM))
        def _():
            mask = ((rows >= s) & (rows < e)).astype(jnp.float32)
            Mn = 0.6 * p_ref[pl.ds(n * K, K), :] + 0.4 * G
            out_ref[...] += 0.2 * jnp.dot(
                W * mask, Mn, preferred_element_type=jnp.float32)


def _run_mix(offs, w_sorted, f_sorted, p_flat, g):
    grid_spec = pltpu.PrefetchScalarGridSpec(
        num_scalar_prefetch=1,
        grid=(NBLK,),
        in_specs=[
            pl.BlockSpec((BM, K), lambda i, offs: (i, 0)),
            pl.BlockSpec((BM, D), lambda i, offs: (i, 0)),
            pl.BlockSpec((ND * K, D), lambda i, offs: (0, 0)),
            pl.BlockSpec((K, D), lambda i, offs: (0, 0)),
        ],
        out_specs=pl.BlockSpec((BM, D), lambda i, offs: (i, 0)),
    )
    return pl.pallas_call(
        _mix_body,
        grid_spec=grid_spec,
        out_shape=jax.ShapeDtypeStruct((B, D), jnp.float32),
    )(offs, w_sorted, f_sorted, p_flat, g)


@functools.cache
def _sc_gather():
    mesh = plsc.VectorSubcoreMesh(core_axis_name="c", subcore_axis_name="s")

    @functools.partial(
        pl.kernel, mesh=mesh,
        out_type=(jax.ShapeDtypeStruct((B, K), jnp.float32),
                  jax.ShapeDtypeStruct((B, D), jnp.float32)),
        scratch_types=[pltpu.VMEM((RPW,), jnp.int32),
                       pltpu.VMEM((RPW, K), jnp.float32),
                       pltpu.VMEM((RPW, D), jnp.float32),
                       pltpu.SemaphoreType.DMA],
    )
    def gather(w_hbm, f_hbm, perm_hbm, wout_hbm, fout_hbm,
               idx_v, wrows_v, frows_v, sem):
        wid = lax.axis_index("s") * 2 + lax.axis_index("c")
        base = wid * RPW
        pltpu.sync_copy(perm_hbm.at[pl.ds(base, RPW)], idx_v)
        pltpu.async_copy(w_hbm.at[idx_v], wrows_v, sem).wait()
        pltpu.sync_copy(wrows_v, wout_hbm.at[pl.ds(base, RPW)])
        pltpu.async_copy(f_hbm.at[idx_v], frows_v, sem).wait()
        pltpu.sync_copy(frows_v, fout_hbm.at[pl.ds(base, RPW)])

    return gather


@functools.cache
def _sc_scatter():
    mesh = plsc.VectorSubcoreMesh(core_axis_name="c", subcore_axis_name="s")

    @functools.partial(
        pl.kernel, mesh=mesh,
        out_type=jax.ShapeDtypeStruct((B, D), jnp.float32),
        scratch_types=[pltpu.VMEM((RPW,), jnp.int32),
                       pltpu.VMEM((RPW, D), jnp.float32),
                       pltpu.SemaphoreType.DMA],
    )
    def scatter(enh_hbm, perm_hbm, out_hbm, idx_v, rows_v, sem):
        wid = lax.axis_index("s") * 2 + lax.axis_index("c")
        base = wid * RPW
        pltpu.sync_copy(perm_hbm.at[pl.ds(base, RPW)], idx_v)
        pltpu.sync_copy(enh_hbm.at[pl.ds(base, RPW)], rows_v)
        pltpu.async_copy(rows_v, out_hbm.at[idx_v], sem).wait()

    return scatter


def kernel(features, domain_ids, ph_W1, ph_b1, ln1_g, ln1_b, ph_W2, ph_b2,
           pw_W1, pw_b1, pw_ln_g, pw_ln_b, pw_W2, pw_b2,
           domain_prototypes, global_prototypes):
    did = jnp.minimum(domain_ids, ND - 1).astype(jnp.int32)
    perm = jnp.argsort(did).astype(jnp.int32)
    counts = jnp.bincount(did, length=ND)
    offs = jnp.concatenate(
        [jnp.zeros((1,), jnp.int32), jnp.cumsum(counts).astype(jnp.int32)])

    feats, w = _run_heads(features, ph_W1, ph_b1, ln1_g, ln1_b, ph_W2, ph_b2,
                          pw_W1, pw_b1, pw_ln_g, pw_ln_b, pw_W2, pw_b2)
    w_sorted, f_sorted = _sc_gather()(w, feats, perm)
    enh_sorted = _run_mix(offs, w_sorted, f_sorted,
                          domain_prototypes.reshape(ND * K, D),
                          global_prototypes)
    enhanced = _sc_scatter()(enh_sorted, perm)
    return (enhanced, w)


# speedup vs baseline: 3.0084x; 2.3066x over previous
import functools

import jax
import jax.numpy as jnp
from jax import lax
from jax.experimental import pallas as pl
from jax.experimental.pallas import tpu as pltpu

B = 2048
D = 256
K = 1024
ND = 8
H = D // 2

BM = 256
NBLK = B // BM


def _gelu_exact(x):
    return 0.5 * x * (1.0 + lax.erf(x * (2.0 ** -0.5)))


def _layernorm(x, g, b):
    mu = jnp.mean(x, axis=-1, keepdims=True)
    var = jnp.mean((x - mu) ** 2, axis=-1, keepdims=True)
    return (x - mu) / jnp.sqrt(var + 1e-5) * g + b


def _fused_body(x_ref, did_ref, w1_ref, b1_ref, g1_ref, be1_ref, w2_ref,
                b2_ref, pw1_ref, pb1_ref, pg_ref, pbe_ref, pw2_ref, pb2_ref,
                p_ref, gp_ref, enh_ref, w_ref):
    x = x_ref[...]
    dot = functools.partial(jnp.dot, preferred_element_type=jnp.float32)
    h = dot(x, w1_ref[...]) + b1_ref[...]
    h = _layernorm(h, g1_ref[...], be1_ref[...])
    h = _gelu_exact(h)
    h = dot(h, w2_ref[...]) + b2_ref[...]
    nrm = jnp.sqrt(jnp.sum(h * h, axis=-1, keepdims=True))
    feats = h / jnp.maximum(nrm, 1e-12)
    t = dot(x, pw1_ref[...]) + pb1_ref[...]
    t = _layernorm(t, pg_ref[...], pbe_ref[...])
    t = _gelu_exact(t)
    logits = dot(t, pw2_ref[...]) + pb2_ref[...]
    m = jnp.max(logits, axis=-1, keepdims=True)
    e = jnp.exp(logits - m)
    w = e / jnp.sum(e, axis=-1, keepdims=True)
    w_ref[...] = w
    did = did_ref[0, 0, :].reshape(BM, 1)
    acc = feats + 0.08 * dot(w, gp_ref[...])
    for n in range(ND):
        sel = (did == n).astype(jnp.float32)
        acc += (0.12 * sel) * dot(w, p_ref[pl.ds(n * K, K), :])
    enh_ref[...] = acc


def kernel(features, domain_ids, ph_W1, ph_b1, ln1_g, ln1_b, ph_W2, ph_b2,
           pw_W1, pw_b1, pw_ln_g, pw_ln_b, pw_W2, pw_b2,
           domain_prototypes, global_prototypes):
    did = jnp.minimum(domain_ids, ND - 1).astype(jnp.int32)
    did3 = did.reshape(NBLK, 1, BM)

    def const(shape):
        return pl.BlockSpec(shape, lambda i: (0,) * len(shape))

    enhanced, w = pl.pallas_call(
        _fused_body,
        grid=(NBLK,),
        in_specs=[
            pl.BlockSpec((BM, D), lambda i: (i, 0)),
            pl.BlockSpec((1, 1, BM), lambda i: (i, 0, 0)),
            const((D, D)), const((1, D)), const((1, D)), const((1, D)),
            const((D, D)), const((1, D)),
            const((D, H)), const((1, H)), const((1, H)), const((1, H)),
            const((H, K)), const((1, K)),
            const((ND * K, D)), const((K, D)),
        ],
        out_specs=[
            pl.BlockSpec((BM, D), lambda i: (i, 0)),
            pl.BlockSpec((BM, K), lambda i: (i, 0)),
        ],
        out_shape=[
            jax.ShapeDtypeStruct((B, D), jnp.float32),
            jax.ShapeDtypeStruct((B, K), jnp.float32),
        ],
    )(features, did3, ph_W1, ph_b1.reshape(1, D), ln1_g.reshape(1, D),
      ln1_b.reshape(1, D), ph_W2, ph_b2.reshape(1, D),
      pw_W1, pw_b1.reshape(1, H), pw_ln_g.reshape(1, H),
      pw_ln_b.reshape(1, H), pw_W2, pw_b2.reshape(1, K),
      domain_prototypes.reshape(ND * K, D), global_prototypes)
    return (enhanced, w)
